# baseline (speedup 1.0000x reference)
import functools

import jax
import jax.numpy as jnp
from jax import lax
from jax.experimental import pallas as pl
from jax.experimental.pallas import tpu as pltpu
from jax.experimental.pallas import tpu_sc as plsc

N = 10000
E = 320000
D = 128
H = 128
L = 3
C = 2

NC = 2
NS = 16
NW = NC * NS
CHUNK = 128
ECHUNKS = E // CHUNK
MAINC = ECHUNKS // NW
XTRA = ECHUNKS - NW * MAINC
SLOTS = 80
NPASS = 2
PCHUNK = SLOTS // NPASS
P1C = MAINC - PCHUNK
RPT = 624
RTAIL = N - NS * RPT

BN = 2000
NB = N // BN

_PREC = lax.Precision.HIGHEST


def _mm(a, b):
    return lax.dot_general(a, b, (((1,), (0,)), ((), ())),
                           preferred_element_type=jnp.float32,
                           precision=_PREC)


def _mmT(a, b):
    return lax.dot_general(a, b, (((1,), (1,)), ((), ())),
                           preferred_element_type=jnp.float32,
                           precision=_PREC)


def _sc_segment_sum(m, src3, dst3, xsrc, xdst, zeros):
    mesh = plsc.VectorSubcoreMesh(core_axis_name="c", subcore_axis_name="s")

    @functools.partial(
        pl.kernel,
        out_type=jax.ShapeDtypeStruct((NC, N, H), jnp.float32),
        mesh=mesh,
        scratch_types=[
            pltpu.VMEM((PCHUNK, CHUNK), jnp.int32),
            pltpu.VMEM((PCHUNK, CHUNK), jnp.int32),
            pltpu.VMEM((CHUNK, H), jnp.float32),
            pltpu.VMEM((CHUNK, H), jnp.float32),
            pltpu.VMEM_SHARED((N, H), jnp.float32),
            pltpu.SemaphoreType.DMA,
            pltpu.SemaphoreType.DMA,
        ],
    )
    def k(m_hbm, src_hbm, dst_hbm, xsrc_hbm, xdst_hbm, z_hbm, out_hbm,
          src_v, dst_v, rows0_v, rows1_v, acc_sh, sem0, sem1):
        c = lax.axis_index("c")
        s = lax.axis_index("s")
        wid = c * NS + s
        pltpu.sync_copy(z_hbm.at[pl.ds(s * RPT, RPT)],
                        acc_sh.at[pl.ds(s * RPT, RPT)])

        @pl.when(s == NS - 1)
        def _():
            pltpu.sync_copy(z_hbm.at[pl.ds(NS * RPT, RTAIL)],
                            acc_sh.at[pl.ds(NS * RPT, RTAIL)])

        plsc.subcore_barrier()

        def run_pass(n_pairs):
            pltpu.async_copy(m_hbm.at[src_v.at[0]], rows0_v, sem0)

            @pl.loop(0, n_pairs)
            def _(kk):
                k2 = 2 * kk
                pltpu.async_copy(m_hbm.at[src_v.at[k2 + 1]], rows1_v, sem1)
                pltpu.make_async_copy(m_hbm.at[src_v.at[0]], rows0_v,
                                      sem0).wait()
                pltpu.sync_copy(rows0_v, acc_sh.at[dst_v.at[k2]], add=True)

                @pl.when(kk + 1 < n_pairs)
                def _():
                    pltpu.async_copy(m_hbm.at[src_v.at[k2 + 2]], rows0_v,
                                     sem0)

                pltpu.make_async_copy(m_hbm.at[src_v.at[0]], rows1_v,
                                      sem1).wait()
                pltpu.sync_copy(rows1_v, acc_sh.at[dst_v.at[k2 + 1]],
                                add=True)

        pltpu.sync_copy(src_hbm.at[wid, pl.ds(0, PCHUNK)], src_v)
        pltpu.sync_copy(dst_hbm.at[wid, pl.ds(0, PCHUNK)], dst_v)
        run_pass(PCHUNK // 2)
        pltpu.sync_copy(src_hbm.at[wid, pl.ds(PCHUNK, PCHUNK)], src_v)
        pltpu.sync_copy(dst_hbm.at[wid, pl.ds(PCHUNK, PCHUNK)], dst_v)
        run_pass(P1C // 2)

        @pl.when(wid < XTRA)
        def _():
            pltpu.sync_copy(xsrc_hbm.at[wid], src_v.at[0])
            pltpu.sync_copy(xdst_hbm.at[wid], dst_v.at[0])
            pltpu.async_copy(m_hbm.at[src_v.at[0]], rows0_v, sem0).wait()
            pltpu.sync_copy(rows0_v, acc_sh.at[dst_v.at[0]], add=True)

        plsc.subcore_barrier()
        pltpu.sync_copy(acc_sh.at[pl.ds(s * RPT, RPT)],
                        out_hbm.at[c, pl.ds(s * RPT, RPT)])

        @pl.when(s == NS - 1)
        def _():
            pltpu.sync_copy(acc_sh.at[pl.ds(NS * RPT, RTAIL)],
                            out_hbm.at[c, pl.ds(NS * RPT, RTAIL)])

    return k(m, src3, dst3, xsrc, xdst, zeros)


def _pre_body(x_ref, win_ref, bin_ref, wg_ref, h_ref, m_ref):
    h = _mmT(x_ref[...], win_ref[...]) + bin_ref[...]
    h_ref[...] = h
    m_ref[...] = _mm(h, wg_ref[...])


def _gru(p_ref, h, wih_ref, bih_ref, whh_ref, bhh_ref):
    agg = p_ref[0] + p_ref[1]
    gi = _mmT(agg, wih_ref[...]) + bih_ref[...]
    gh = _mmT(h, whh_ref[...]) + bhh_ref[...]
    r = jax.nn.sigmoid(gi[:, :H] + gh[:, :H])
    z = jax.nn.sigmoid(gi[:, H:2 * H] + gh[:, H:2 * H])
    n = jnp.tanh(gi[:, 2 * H:] + r * gh[:, 2 * H:])
    return (1.0 - z) * n + z * h


def _mid_body(p_ref, h_ref, wih_ref, bih_ref, whh_ref, bhh_ref, wg_ref,
              h1_ref, m1_ref):
    h1 = _gru(p_ref, h_ref[...], wih_ref, bih_ref, whh_ref, bhh_ref)
    h1_ref[...] = h1
    m1_ref[...] = _mm(h1, wg_ref[...])


def _post_body(p_ref, h_ref, wih_ref, bih_ref, whh_ref, bhh_ref, w1_ref,
               b1_ref, w2_ref, b2_ref, out_ref, sum_sc, max_sc):
    i = pl.program_id(0)
    h1 = _gru(p_ref, h_ref[...], wih_ref, bih_ref, whh_ref, bhh_ref)
    bsum = jnp.sum(h1, axis=0, keepdims=True)
    bmax = jnp.max(h1, axis=0, keepdims=True)

    @pl.when(i == 0)
    def _():
        sum_sc[...] = bsum
        max_sc[...] = bmax

    @pl.when(i > 0)
    def _():
        sum_sc[...] += bsum
        max_sc[...] = jnp.maximum(max_sc[...], bmax)

    @pl.when(i == NB - 1)
    def _():
        feat = jnp.concatenate([sum_sc[...] / N, max_sc[...]], axis=1)
        hid = jax.nn.relu(_mmT(feat, w1_ref[...]) + b1_ref[...])
        out_ref[...] = _mmT(hid, w2_ref[...]) + b2_ref[...]


def _row_spec(width):
    return pl.BlockSpec((BN, width), lambda i: (i, 0))


def _full_spec(shape):
    return pl.BlockSpec(shape, lambda i: tuple(0 for _ in shape))


def kernel(x, edge_index, W_in, b_in, ggc_w, gru_wih, gru_whh, gru_bih,
           gru_bhh, W1, b1, W2, b2):
    nmain = NW * MAINC * CHUNK
    src3 = jnp.pad(
        edge_index[0][:nmain].reshape(NW, MAINC, CHUNK),
        ((0, 0), (0, SLOTS - MAINC), (0, 0)))
    dst3 = jnp.pad(
        edge_index[1][:nmain].reshape(NW, MAINC, CHUNK),
        ((0, 0), (0, SLOTS - MAINC), (0, 0)))
    xsrc = edge_index[0][nmain:].reshape(XTRA, CHUNK)
    xdst = edge_index[1][nmain:].reshape(XTRA, CHUNK)
    zeros = jnp.zeros((N, H), jnp.float32)
    b_in2 = b_in.reshape(1, H)
    bih2 = gru_bih.reshape(1, 3 * H)
    bhh2 = gru_bhh.reshape(1, 3 * H)
    b1_2 = b1.reshape(1, H)
    b2_2 = b2.reshape(1, C)

    h, m = pl.pallas_call(
        _pre_body,
        grid=(NB,),
        in_specs=[_row_spec(D)] + [_full_spec(s)
                                   for s in ((H, D), (1, H), (H, H))],
        out_specs=[_row_spec(H), _row_spec(H)],
        out_shape=[jax.ShapeDtypeStruct((N, H), jnp.float32),
                   jax.ShapeDtypeStruct((N, H), jnp.float32)],
    )(x, W_in, b_in2, ggc_w[0])

    gru_w_specs = [_full_spec(s) for s in
                   ((3 * H, H), (1, 3 * H), (3 * H, H), (1, 3 * H))]
    p_spec = pl.BlockSpec((NC, BN, H), lambda i: (0, i, 0))
    for r in range(L - 1):
        p = _sc_segment_sum(m, src3, dst3, xsrc, xdst, zeros)
        h, m = pl.pallas_call(
            _mid_body,
            grid=(NB,),
            in_specs=[p_spec, _row_spec(H)] + gru_w_specs
            + [_full_spec((H, H))],
            out_specs=[_row_spec(H), _row_spec(H)],
            out_shape=[jax.ShapeDtypeStruct((N, H), jnp.float32),
                       jax.ShapeDtypeStruct((N, H), jnp.float32)],
        )(p, h, gru_wih, bih2, gru_whh, bhh2, ggc_w[r + 1])

    p = _sc_segment_sum(m, src3, dst3, xsrc, xdst, zeros)
    out = pl.pallas_call(
        _post_body,
        grid=(NB,),
        in_specs=[p_spec, _row_spec(H)] + gru_w_specs
        + [_full_spec(s) for s in ((H, 2 * H), (1, H), (C, H), (1, C))],
        out_specs=pl.BlockSpec((1, C), lambda i: (0, 0)),
        out_shape=jax.ShapeDtypeStruct((1, C), jnp.float32),
        scratch_shapes=[pltpu.VMEM((1, H), jnp.float32),
                        pltpu.VMEM((1, H), jnp.float32)],
    )(p, h, gru_wih, bih2, gru_whh, bhh2, W1, b1_2, W2, b2_2)
    return out


# speedup vs baseline: 1.1950x; 1.1950x over previous
import functools

import jax
import jax.numpy as jnp
from jax import lax
from jax.experimental import pallas as pl
from jax.experimental.pallas import tpu as pltpu
from jax.experimental.pallas import tpu_sc as plsc

N = 10000
E = 320000
D = 128
H = 128
L = 3
C = 2

NC = 2
NS = 16
NW = NC * NS
CHUNK = 128
ECHUNKS = E // CHUNK
MAINC = ECHUNKS // NW
XTRA = ECHUNKS - NW * MAINC
SLOTS = 80
NPASS = 2
PCHUNK = SLOTS // NPASS
P1C = MAINC - PCHUNK
RPT = 624
RTAIL = N - NS * RPT

BN = 2000
NB = N // BN

_PREC = lax.Precision.DEFAULT


def _mm(a, b):
    return lax.dot_general(a, b, (((1,), (0,)), ((), ())),
                           preferred_element_type=jnp.float32,
                           precision=_PREC)


def _mmT(a, b):
    return lax.dot_general(a, b, (((1,), (1,)), ((), ())),
                           preferred_element_type=jnp.float32,
                           precision=_PREC)


def _sc_segment_sum(m, src3, dst3, xsrc, xdst, zeros):
    mesh = plsc.VectorSubcoreMesh(core_axis_name="c", subcore_axis_name="s")

    @functools.partial(
        pl.kernel,
        out_type=jax.ShapeDtypeStruct((NC, N, H), jnp.float32),
        mesh=mesh,
        scratch_types=[
            pltpu.VMEM((PCHUNK, CHUNK), jnp.int32),
            pltpu.VMEM((PCHUNK, CHUNK), jnp.int32),
            pltpu.VMEM((CHUNK, H), jnp.float32),
            pltpu.VMEM((CHUNK, H), jnp.float32),
            pltpu.VMEM_SHARED((N, H), jnp.float32),
            pltpu.SemaphoreType.DMA,
            pltpu.SemaphoreType.DMA,
        ],
    )
    def k(m_hbm, src_hbm, dst_hbm, xsrc_hbm, xdst_hbm, z_hbm, out_hbm,
          src_v, dst_v, rows0_v, rows1_v, acc_sh, sem0, sem1):
        c = lax.axis_index("c")
        s = lax.axis_index("s")
        wid = c * NS + s
        pltpu.sync_copy(z_hbm.at[pl.ds(s * RPT, RPT)],
                        acc_sh.at[pl.ds(s * RPT, RPT)])

        @pl.when(s == NS - 1)
        def _():
            pltpu.sync_copy(z_hbm.at[pl.ds(NS * RPT, RTAIL)],
                            acc_sh.at[pl.ds(NS * RPT, RTAIL)])

        plsc.subcore_barrier()

        def run_pass(n_pairs):
            pltpu.async_copy(m_hbm.at[src_v.at[0]], rows0_v, sem0)

            @pl.loop(0, n_pairs)
            def _(kk):
                k2 = 2 * kk
                pltpu.async_copy(m_hbm.at[src_v.at[k2 + 1]], rows1_v, sem1)
                pltpu.make_async_copy(m_hbm.at[src_v.at[0]], rows0_v,
                                      sem0).wait()
                pltpu.sync_copy(rows0_v, acc_sh.at[dst_v.at[k2]], add=True)

                @pl.when(kk + 1 < n_pairs)
                def _():
                    pltpu.async_copy(m_hbm.at[src_v.at[k2 + 2]], rows0_v,
                                     sem0)

                pltpu.make_async_copy(m_hbm.at[src_v.at[0]], rows1_v,
                                      sem1).wait()
                pltpu.sync_copy(rows1_v, acc_sh.at[dst_v.at[k2 + 1]],
                                add=True)

        pltpu.sync_copy(src_hbm.at[wid, pl.ds(0, PCHUNK)], src_v)
        pltpu.sync_copy(dst_hbm.at[wid, pl.ds(0, PCHUNK)], dst_v)
        run_pass(PCHUNK // 2)
        pltpu.sync_copy(src_hbm.at[wid, pl.ds(PCHUNK, PCHUNK)], src_v)
        pltpu.sync_copy(dst_hbm.at[wid, pl.ds(PCHUNK, PCHUNK)], dst_v)
        run_pass(P1C // 2)

        @pl.when(wid < XTRA)
        def _():
            pltpu.sync_copy(xsrc_hbm.at[wid], src_v.at[0])
            pltpu.sync_copy(xdst_hbm.at[wid], dst_v.at[0])
            pltpu.async_copy(m_hbm.at[src_v.at[0]], rows0_v, sem0).wait()
            pltpu.sync_copy(rows0_v, acc_sh.at[dst_v.at[0]], add=True)

        plsc.subcore_barrier()
        pltpu.sync_copy(acc_sh.at[pl.ds(s * RPT, RPT)],
                        out_hbm.at[c, pl.ds(s * RPT, RPT)])

        @pl.when(s == NS - 1)
        def _():
            pltpu.sync_copy(acc_sh.at[pl.ds(NS * RPT, RTAIL)],
                            out_hbm.at[c, pl.ds(NS * RPT, RTAIL)])

    return k(m, src3, dst3, xsrc, xdst, zeros)


def _pre_body(x_ref, win_ref, bin_ref, wg_ref, h_ref, m_ref):
    h = _mmT(x_ref[...], win_ref[...]) + bin_ref[...]
    h_ref[...] = h
    m_ref[...] = _mm(h, wg_ref[...])


def _gru(p_ref, h, wih_ref, bih_ref, whh_ref, bhh_ref):
    agg = p_ref[0] + p_ref[1]
    gi = _mmT(agg, wih_ref[...]) + bih_ref[...]
    gh = _mmT(h, whh_ref[...]) + bhh_ref[...]
    r = jax.nn.sigmoid(gi[:, :H] + gh[:, :H])
    z = jax.nn.sigmoid(gi[:, H:2 * H] + gh[:, H:2 * H])
    n = jnp.tanh(gi[:, 2 * H:] + r * gh[:, 2 * H:])
    return (1.0 - z) * n + z * h


def _mid_body(p_ref, h_ref, wih_ref, bih_ref, whh_ref, bhh_ref, wg_ref,
              h1_ref, m1_ref):
    h1 = _gru(p_ref, h_ref[...], wih_ref, bih_ref, whh_ref, bhh_ref)
    h1_ref[...] = h1
    m1_ref[...] = _mm(h1, wg_ref[...])


def _post_body(p_ref, h_ref, wih_ref, bih_ref, whh_ref, bhh_ref, w1_ref,
               b1_ref, w2_ref, b2_ref, out_ref, sum_sc, max_sc):
    i = pl.program_id(0)
    h1 = _gru(p_ref, h_ref[...], wih_ref, bih_ref, whh_ref, bhh_ref)
    bsum = jnp.sum(h1, axis=0, keepdims=True)
    bmax = jnp.max(h1, axis=0, keepdims=True)

    @pl.when(i == 0)
    def _():
        sum_sc[...] = bsum
        max_sc[...] = bmax

    @pl.when(i > 0)
    def _():
        sum_sc[...] += bsum
        max_sc[...] = jnp.maximum(max_sc[...], bmax)

    @pl.when(i == NB - 1)
    def _():
        feat = jnp.concatenate([sum_sc[...] / N, max_sc[...]], axis=1)
        hid = jax.nn.relu(_mmT(feat, w1_ref[...]) + b1_ref[...])
        out_ref[...] = _mmT(hid, w2_ref[...]) + b2_ref[...]


def _row_spec(width):
    return pl.BlockSpec((BN, width), lambda i: (i, 0))


def _full_spec(shape):
    return pl.BlockSpec(shape, lambda i: tuple(0 for _ in shape))


def kernel(x, edge_index, W_in, b_in, ggc_w, gru_wih, gru_whh, gru_bih,
           gru_bhh, W1, b1, W2, b2):
    nmain = NW * MAINC * CHUNK
    src3 = jnp.pad(
        edge_index[0][:nmain].reshape(NW, MAINC, CHUNK),
        ((0, 0), (0, SLOTS - MAINC), (0, 0)))
    dst3 = jnp.pad(
        edge_index[1][:nmain].reshape(NW, MAINC, CHUNK),
        ((0, 0), (0, SLOTS - MAINC), (0, 0)))
    xsrc = edge_index[0][nmain:].reshape(XTRA, CHUNK)
    xdst = edge_index[1][nmain:].reshape(XTRA, CHUNK)
    zeros = jnp.zeros((N, H), jnp.float32)
    b_in2 = b_in.reshape(1, H)
    bih2 = gru_bih.reshape(1, 3 * H)
    bhh2 = gru_bhh.reshape(1, 3 * H)
    b1_2 = b1.reshape(1, H)
    b2_2 = b2.reshape(1, C)

    h, m = pl.pallas_call(
        _pre_body,
        grid=(NB,),
        in_specs=[_row_spec(D)] + [_full_spec(s)
                                   for s in ((H, D), (1, H), (H, H))],
        out_specs=[_row_spec(H), _row_spec(H)],
        out_shape=[jax.ShapeDtypeStruct((N, H), jnp.float32),
                   jax.ShapeDtypeStruct((N, H), jnp.float32)],
    )(x, W_in, b_in2, ggc_w[0])

    gru_w_specs = [_full_spec(s) for s in
                   ((3 * H, H), (1, 3 * H), (3 * H, H), (1, 3 * H))]
    p_spec = pl.BlockSpec((NC, BN, H), lambda i: (0, i, 0))
    for r in range(L - 1):
        p = _sc_segment_sum(m, src3, dst3, xsrc, xdst, zeros)
        h, m = pl.pallas_call(
            _mid_body,
            grid=(NB,),
            in_specs=[p_spec, _row_spec(H)] + gru_w_specs
            + [_full_spec((H, H))],
            out_specs=[_row_spec(H), _row_spec(H)],
            out_shape=[jax.ShapeDtypeStruct((N, H), jnp.float32),
                       jax.ShapeDtypeStruct((N, H), jnp.float32)],
        )(p, h, gru_wih, bih2, gru_whh, bhh2, ggc_w[r + 1])

    p = _sc_segment_sum(m, src3, dst3, xsrc, xdst, zeros)
    out = pl.pallas_call(
        _post_body,
        grid=(NB,),
        in_specs=[p_spec, _row_spec(H)] + gru_w_specs
        + [_full_spec(s) for s in ((H, 2 * H), (1, H), (C, H), (1, C))],
        out_specs=pl.BlockSpec((1, C), lambda i: (0, 0)),
        out_shape=jax.ShapeDtypeStruct((1, C), jnp.float32),
        scratch_shapes=[pltpu.VMEM((1, H), jnp.float32),
                        pltpu.VMEM((1, H), jnp.float32)],
    )(p, h, gru_wih, bih2, gru_whh, bhh2, W1, b1_2, W2, b2_2)
    return out


# speedup vs baseline: 1.2219x; 1.0225x over previous
import functools

import jax
import jax.numpy as jnp
from jax import lax
from jax.experimental import pallas as pl
from jax.experimental.pallas import tpu as pltpu
from jax.experimental.pallas import tpu_sc as plsc

N = 10000
E = 320000
D = 128
H = 128
L = 3
C = 2

NC = 2
NS = 16
NW = NC * NS
CHUNK = 128
ECHUNKS = E // CHUNK
MAINC = ECHUNKS // NW
XTRA = ECHUNKS - NW * MAINC
SLOTS = 80
NPASS = 2
PCHUNK = SLOTS // NPASS
P1C = MAINC - PCHUNK
RPT = 624
RTAIL = N - NS * RPT

BN = 2000
NB = N // BN

_PREC = lax.Precision.DEFAULT


def _mm(a, b):
    return lax.dot_general(a, b, (((1,), (0,)), ((), ())),
                           preferred_element_type=jnp.float32,
                           precision=_PREC)


def _mmT(a, b):
    return lax.dot_general(a, b, (((1,), (1,)), ((), ())),
                           preferred_element_type=jnp.float32,
                           precision=_PREC)


def _sc_segment_sum(m, src3, dst3, xsrc, xdst, zeros):
    mesh = plsc.VectorSubcoreMesh(core_axis_name="c", subcore_axis_name="s")

    @functools.partial(
        pl.kernel,
        out_type=jax.ShapeDtypeStruct((NC, N, H), jnp.float32),
        mesh=mesh,
        scratch_types=[
            pltpu.VMEM((PCHUNK, CHUNK), jnp.int32),
            pltpu.VMEM((PCHUNK, CHUNK), jnp.int32),
            pltpu.VMEM((CHUNK, H), jnp.float32),
            pltpu.VMEM((CHUNK, H), jnp.float32),
            pltpu.VMEM_SHARED((N, H), jnp.float32),
            pltpu.SemaphoreType.DMA,
            pltpu.SemaphoreType.DMA,
            pltpu.SemaphoreType.DMA,
        ],
    )
    def k(m_hbm, src_hbm, dst_hbm, xsrc_hbm, xdst_hbm, z_hbm, out_hbm,
          src_v, dst_v, rows0_v, rows1_v, acc_sh, sem0, sem1, semz):
        c = lax.axis_index("c")
        s = lax.axis_index("s")
        wid = c * NS + s
        pltpu.async_copy(z_hbm.at[pl.ds(s * RPT, RPT)],
                         acc_sh.at[pl.ds(s * RPT, RPT)], semz)

        @pl.when(s == NS - 1)
        def _():
            pltpu.async_copy(z_hbm.at[pl.ds(NS * RPT, RTAIL)],
                             acc_sh.at[pl.ds(NS * RPT, RTAIL)], semz)

        def run_pass(n_pairs):
            @pl.loop(0, n_pairs)
            def _(kk):
                k2 = 2 * kk
                pltpu.async_copy(m_hbm.at[src_v.at[k2 + 1]], rows1_v, sem1)
                pltpu.make_async_copy(m_hbm.at[src_v.at[0]], rows0_v,
                                      sem0).wait()
                pltpu.sync_copy(rows0_v, acc_sh.at[dst_v.at[k2]], add=True)

                @pl.when(kk + 1 < n_pairs)
                def _():
                    pltpu.async_copy(m_hbm.at[src_v.at[k2 + 2]], rows0_v,
                                     sem0)

                pltpu.make_async_copy(m_hbm.at[src_v.at[0]], rows1_v,
                                      sem1).wait()
                pltpu.sync_copy(rows1_v, acc_sh.at[dst_v.at[k2 + 1]],
                                add=True)

        pltpu.sync_copy(src_hbm.at[wid, pl.ds(0, PCHUNK)], src_v)
        pltpu.sync_copy(dst_hbm.at[wid, pl.ds(0, PCHUNK)], dst_v)
        pltpu.async_copy(m_hbm.at[src_v.at[0]], rows0_v, sem0)
        pltpu.make_async_copy(z_hbm.at[pl.ds(s * RPT, RPT)],
                              acc_sh.at[pl.ds(s * RPT, RPT)], semz).wait()

        @pl.when(s == NS - 1)
        def _():
            pltpu.make_async_copy(z_hbm.at[pl.ds(NS * RPT, RTAIL)],
                                  acc_sh.at[pl.ds(NS * RPT, RTAIL)],
                                  semz).wait()

        plsc.subcore_barrier()
        run_pass(PCHUNK // 2)
        pltpu.sync_copy(src_hbm.at[wid, pl.ds(PCHUNK, PCHUNK)], src_v)
        pltpu.sync_copy(dst_hbm.at[wid, pl.ds(PCHUNK, PCHUNK)], dst_v)
        pltpu.async_copy(m_hbm.at[src_v.at[0]], rows0_v, sem0)
        run_pass(P1C // 2)

        @pl.when(wid < XTRA)
        def _():
            pltpu.sync_copy(xsrc_hbm.at[wid], src_v.at[0])
            pltpu.sync_copy(xdst_hbm.at[wid], dst_v.at[0])
            pltpu.async_copy(m_hbm.at[src_v.at[0]], rows0_v, sem0).wait()
            pltpu.sync_copy(rows0_v, acc_sh.at[dst_v.at[0]], add=True)

        plsc.subcore_barrier()
        pltpu.sync_copy(acc_sh.at[pl.ds(s * RPT, RPT)],
                        out_hbm.at[c, pl.ds(s * RPT, RPT)])

        @pl.when(s == NS - 1)
        def _():
            pltpu.sync_copy(acc_sh.at[pl.ds(NS * RPT, RTAIL)],
                            out_hbm.at[c, pl.ds(NS * RPT, RTAIL)])

    return k(m, src3, dst3, xsrc, xdst, zeros)


def _pre_body(x_ref, win_ref, bin_ref, wg_ref, h_ref, m_ref):
    h = _mmT(x_ref[...], win_ref[...]) + bin_ref[...]
    h_ref[...] = h
    m_ref[...] = _mm(h, wg_ref[...])


def _gru(p_ref, h, wih_ref, bih_ref, whh_ref, bhh_ref):
    agg = p_ref[0] + p_ref[1]
    gi = _mmT(agg, wih_ref[...]) + bih_ref[...]
    gh = _mmT(h, whh_ref[...]) + bhh_ref[...]
    r = jax.nn.sigmoid(gi[:, :H] + gh[:, :H])
    z = jax.nn.sigmoid(gi[:, H:2 * H] + gh[:, H:2 * H])
    n = jnp.tanh(gi[:, 2 * H:] + r * gh[:, 2 * H:])
    return (1.0 - z) * n + z * h


def _mid_body(p_ref, h_ref, wih_ref, bih_ref, whh_ref, bhh_ref, wg_ref,
              h1_ref, m1_ref):
    h1 = _gru(p_ref, h_ref[...], wih_ref, bih_ref, whh_ref, bhh_ref)
    h1_ref[...] = h1
    m1_ref[...] = _mm(h1, wg_ref[...])


def _post_body(p_ref, h_ref, wih_ref, bih_ref, whh_ref, bhh_ref, w1_ref,
               b1_ref, w2_ref, b2_ref, out_ref, sum_sc, max_sc):
    i = pl.program_id(0)
    h1 = _gru(p_ref, h_ref[...], wih_ref, bih_ref, whh_ref, bhh_ref)
    bsum = jnp.sum(h1, axis=0, keepdims=True)
    bmax = jnp.max(h1, axis=0, keepdims=True)

    @pl.when(i == 0)
    def _():
        sum_sc[...] = bsum
        max_sc[...] = bmax

    @pl.when(i > 0)
    def _():
        sum_sc[...] += bsum
        max_sc[...] = jnp.maximum(max_sc[...], bmax)

    @pl.when(i == NB - 1)
    def _():
        feat = jnp.concatenate([sum_sc[...] / N, max_sc[...]], axis=1)
        hid = jax.nn.relu(_mmT(feat, w1_ref[...]) + b1_ref[...])
        out_ref[...] = _mmT(hid, w2_ref[...]) + b2_ref[...]


def _row_spec(width):
    return pl.BlockSpec((BN, width), lambda i: (i, 0))


def _full_spec(shape):
    return pl.BlockSpec(shape, lambda i: tuple(0 for _ in shape))


def kernel(x, edge_index, W_in, b_in, ggc_w, gru_wih, gru_whh, gru_bih,
           gru_bhh, W1, b1, W2, b2):
    nmain = NW * MAINC * CHUNK
    src3 = jnp.pad(
        edge_index[0][:nmain].reshape(NW, MAINC, CHUNK),
        ((0, 0), (0, SLOTS - MAINC), (0, 0)))
    dst3 = jnp.pad(
        edge_index[1][:nmain].reshape(NW, MAINC, CHUNK),
        ((0, 0), (0, SLOTS - MAINC), (0, 0)))
    xsrc = edge_index[0][nmain:].reshape(XTRA, CHUNK)
    xdst = edge_index[1][nmain:].reshape(XTRA, CHUNK)
    zeros = jnp.zeros((N, H), jnp.float32)
    b_in2 = b_in.reshape(1, H)
    bih2 = gru_bih.reshape(1, 3 * H)
    bhh2 = gru_bhh.reshape(1, 3 * H)
    b1_2 = b1.reshape(1, H)
    b2_2 = b2.reshape(1, C)

    h, m = pl.pallas_call(
        _pre_body,
        grid=(NB,),
        in_specs=[_row_spec(D)] + [_full_spec(s)
                                   for s in ((H, D), (1, H), (H, H))],
        out_specs=[_row_spec(H), _row_spec(H)],
        out_shape=[jax.ShapeDtypeStruct((N, H), jnp.float32),
                   jax.ShapeDtypeStruct((N, H), jnp.float32)],
    )(x, W_in, b_in2, ggc_w[0])

    gru_w_specs = [_full_spec(s) for s in
                   ((3 * H, H), (1, 3 * H), (3 * H, H), (1, 3 * H))]
    p_spec = pl.BlockSpec((NC, BN, H), lambda i: (0, i, 0))
    for r in range(L - 1):
        p = _sc_segment_sum(m, src3, dst3, xsrc, xdst, zeros)
        h, m = pl.pallas_call(
            _mid_body,
            grid=(NB,),
            in_specs=[p_spec, _row_spec(H)] + gru_w_specs
            + [_full_spec((H, H))],
            out_specs=[_row_spec(H), _row_spec(H)],
            out_shape=[jax.ShapeDtypeStruct((N, H), jnp.float32),
                       jax.ShapeDtypeStruct((N, H), jnp.float32)],
        )(p, h, gru_wih, bih2, gru_whh, bhh2, ggc_w[r + 1])

    p = _sc_segment_sum(m, src3, dst3, xsrc, xdst, zeros)
    out = pl.pallas_call(
        _post_body,
        grid=(NB,),
        in_specs=[p_spec, _row_spec(H)] + gru_w_specs
        + [_full_spec(s) for s in ((H, 2 * H), (1, H), (C, H), (1, C))],
        out_specs=pl.BlockSpec((1, C), lambda i: (0, 0)),
        out_shape=jax.ShapeDtypeStruct((1, C), jnp.float32),
        scratch_shapes=[pltpu.VMEM((1, H), jnp.float32),
                        pltpu.VMEM((1, H), jnp.float32)],
    )(p, h, gru_wih, bih2, gru_whh, bhh2, W1, b1_2, W2, b2_2)
    return out
